# baseline (device time: 353400 ns/iter reference)
import jax
import jax.numpy as jnp
from jax import lax
from jax.experimental import pallas as pl
from jax.experimental.pallas import tpu as pltpu

T_LOC = 1024
D = 1024
E_LOC = 8
E_GLB = 16
F = 4096
C = 320
FT = 512


def _peer():
    return (lax.axis_index("x"), 1 - lax.axis_index("y"))


def _peer_barrier():
    bar = pltpu.get_barrier_semaphore()
    pl.semaphore_signal(
        bar, inc=1, device_id=_peer(), device_id_type=pl.DeviceIdType.MESH
    )
    pl.semaphore_wait(bar, 1)


def _ag_x_router(x, router):

    def body(x_ref, r_ref, xf_ref, rs_ref, send_sems, recv_sems):
        my_y = lax.axis_index("y")
        _peer_barrier()
        row0 = my_y * T_LOC
        xf_ref[pl.ds(row0, T_LOC), :] = x_ref[...].astype(jnp.bfloat16)
        rs_ref[my_y] = r_ref[...]
        rdma_x = pltpu.make_async_remote_copy(
            src_ref=xf_ref.at[pl.ds(row0, T_LOC), :],
            dst_ref=xf_ref.at[pl.ds(row0, T_LOC), :],
            send_sem=send_sems.at[0],
            recv_sem=recv_sems.at[0],
            device_id=_peer(),
            device_id_type=pl.DeviceIdType.MESH,
        )
        rdma_r = pltpu.make_async_remote_copy(
            src_ref=rs_ref.at[my_y],
            dst_ref=rs_ref.at[my_y],
            send_sem=send_sems.at[1],
            recv_sem=recv_sems.at[1],
            device_id=_peer(),
            device_id_type=pl.DeviceIdType.MESH,
        )
        rdma_x.start()
        rdma_r.start()
        rdma_r.wait()
        rdma_x.wait()

    return pl.pallas_call(
        body,
        out_shape=(
            jax.ShapeDtypeStruct((2 * T_LOC, D), jnp.bfloat16),
            jax.ShapeDtypeStruct((2, D, E_LOC), jnp.float32),
        ),
        in_specs=[
            pl.BlockSpec(memory_space=pltpu.VMEM),
            pl.BlockSpec(memory_space=pltpu.VMEM),
        ],
        out_specs=(
            pl.BlockSpec(memory_space=pltpu.VMEM),
            pl.BlockSpec(memory_space=pltpu.VMEM),
        ),
        scratch_shapes=[
            pltpu.SemaphoreType.DMA((2,)),
            pltpu.SemaphoreType.DMA((2,)),
        ],
        compiler_params=pltpu.CompilerParams(collective_id=0),
    )(x, router)


def _ag_gates(gates_mine):

    def body(g_ref, gf_ref, send_sem, recv_sem):
        my_y = lax.axis_index("y")
        _peer_barrier()
        row0 = my_y * T_LOC
        gf_ref[pl.ds(row0, T_LOC), :] = g_ref[...]
        rdma = pltpu.make_async_remote_copy(
            src_ref=gf_ref.at[pl.ds(row0, T_LOC), :],
            dst_ref=gf_ref.at[pl.ds(row0, T_LOC), :],
            send_sem=send_sem,
            recv_sem=recv_sem,
            device_id=_peer(),
            device_id_type=pl.DeviceIdType.MESH,
        )
        rdma.start()
        rdma.wait()

    return pl.pallas_call(
        body,
        out_shape=jax.ShapeDtypeStruct((2 * T_LOC, E_GLB), jnp.float32),
        in_specs=[pl.BlockSpec(memory_space=pltpu.VMEM)],
        out_specs=pl.BlockSpec(memory_space=pltpu.VMEM),
        scratch_shapes=[pltpu.SemaphoreType.DMA, pltpu.SemaphoreType.DMA],
        compiler_params=pltpu.CompilerParams(collective_id=1),
    )(gates_mine)


def _expert_ffn(xg, W1, W2):

    def body(xg_ref, w1_ref, w2_ref, out_ref):
        fi = pl.program_id(1)
        xe = xg_ref[0]
        w1 = w1_ref[0].astype(jnp.bfloat16)
        h = jnp.dot(xe, w1, preferred_element_type=jnp.float32)
        h = jnp.maximum(h, 0.0).astype(jnp.bfloat16)
        w2 = w2_ref[0].astype(jnp.bfloat16)
        y = jnp.dot(h, w2, preferred_element_type=jnp.float32)

        @pl.when(fi == 0)
        def _():
            out_ref[...] = jnp.zeros_like(out_ref)

        out_ref[...] += y[None]

    return pl.pallas_call(
        body,
        grid=(E_LOC, F // FT),
        in_specs=[
            pl.BlockSpec((1, C, D), lambda e, fi: (e, 0, 0)),
            pl.BlockSpec((1, D, FT), lambda e, fi: (e, 0, fi)),
            pl.BlockSpec((1, FT, D), lambda e, fi: (e, fi, 0)),
        ],
        out_specs=pl.BlockSpec((1, C, D), lambda e, fi: (e, 0, 0)),
        out_shape=jax.ShapeDtypeStruct((E_LOC, C, D), jnp.float32),
        compiler_params=pltpu.CompilerParams(
            dimension_semantics=("parallel", "arbitrary")
        ),
    )(xg, W1, W2)


def _reduce_scatter(out_full):

    def body(of_ref, out_ref, sbuf, rbuf, send_sem, recv_sem):
        my_y = lax.axis_index("y")
        _peer_barrier()
        sbuf[...] = of_ref[pl.ds((1 - my_y) * T_LOC, T_LOC), :].astype(
            jnp.bfloat16
        )
        rdma = pltpu.make_async_remote_copy(
            src_ref=sbuf,
            dst_ref=rbuf,
            send_sem=send_sem,
            recv_sem=recv_sem,
            device_id=_peer(),
            device_id_type=pl.DeviceIdType.MESH,
        )
        rdma.start()
        rdma.wait()
        out_ref[...] = of_ref[pl.ds(my_y * T_LOC, T_LOC), :] + rbuf[
            ...
        ].astype(jnp.float32)

    return pl.pallas_call(
        body,
        out_shape=jax.ShapeDtypeStruct((T_LOC, D), jnp.float32),
        in_specs=[pl.BlockSpec(memory_space=pltpu.VMEM)],
        out_specs=pl.BlockSpec(memory_space=pltpu.VMEM),
        scratch_shapes=[
            pltpu.VMEM((T_LOC, D), jnp.bfloat16),
            pltpu.VMEM((T_LOC, D), jnp.bfloat16),
            pltpu.SemaphoreType.DMA,
            pltpu.SemaphoreType.DMA,
        ],
        compiler_params=pltpu.CompilerParams(collective_id=2),
    )(out_full)


def kernel(x, router, W1, W2):
    my_y = lax.axis_index("y")

    x_full, r_stack = _ag_x_router(x, router)
    router_full = jnp.moveaxis(r_stack, 0, 1).reshape(D, E_GLB)

    gates_mine = jnp.dot(x, router_full, precision=lax.Precision.HIGHEST)
    gates_full = _ag_gates(gates_mine)

    topv, topi = lax.top_k(gates_full, 2)
    w = jax.nn.softmax(topv, axis=-1)
    eg = my_y * E_LOC + jnp.arange(E_LOC)
    hit = topi[:, :, None] == eg[None, None, :]
    mask = hit.any(axis=1)
    tok_w = jnp.sum(jnp.where(hit, w[:, :, None], 0.0), axis=1)

    pos = jnp.cumsum(mask.astype(jnp.int32), axis=0) - 1
    el = jnp.arange(E_LOC)[None, :]
    slot = jnp.where(mask & (pos < C), el * C + pos, E_LOC * C)
    tok_ids = jnp.broadcast_to(
        jnp.arange(2 * T_LOC, dtype=jnp.int32)[:, None], slot.shape
    )
    idx = (
        jnp.zeros((E_LOC * C,), jnp.int32)
        .at[slot.reshape(-1)]
        .set(tok_ids.reshape(-1), mode="drop")
    )
    wslot = (
        jnp.zeros((E_LOC * C,), jnp.float32)
        .at[slot.reshape(-1)]
        .set(tok_w.reshape(-1), mode="drop")
    )

    xg = x_full[idx].reshape(E_LOC, C, D)
    yg = _expert_ffn(xg, W1, W2)

    contrib = yg.reshape(E_LOC * C, D) * wslot[:, None]
    out_full = (
        jnp.zeros((2 * T_LOC, D), jnp.float32).at[idx].add(contrib)
    )
    return _reduce_scatter(out_full)
